# baseline (speedup 1.0000x reference)
import math

import jax
import jax.numpy as jnp
from jax import lax
from jax.experimental import pallas as pl
from jax.experimental.pallas import tpu as pltpu
from jax.experimental.pallas import tpu_sc as plsc

D = 64
SCALE = math.sqrt(D)
NC = 1
NS = 16
NW = NC * NS
C = 128
NBUF = 4
LANES = 16


def _emb_body(x_hbm, table_hbm, out_hbm, idx_v, rows_v, *sems):
    sem_g = sems[:NBUF]
    sem_s = sems[NBUF:]
    n_chunks = x_hbm.shape[1]
    b_per_w = n_chunks * C
    wid = lax.axis_index("s") * NC + lax.axis_index("c")
    base = wid * b_per_w

    def out_slice(g):
        return out_hbm.at[pl.ds(base + g * C, C)]

    def gather(g, b, sem):
        return pltpu.make_async_copy(
            table_hbm.at[idx_v.at[g]], rows_v.at[b], sem)

    def scatter(g, b, sem):
        return pltpu.make_async_copy(rows_v.at[b], out_slice(g), sem)

    pltpu.sync_copy(x_hbm.at[wid], idx_v)

    for b in range(NBUF):
        gather(b, b, sem_g[b]).start()

    @pl.loop(0, n_chunks // NBUF)
    def _outer(t):
        g0 = t * NBUF
        for bb in range(NBUF):
            g = g0 + bb
            pb = (bb - 1) % NBUF
            p = g - 1
            nxt = p + NBUF

            @pl.when(jnp.logical_and(p >= 0, nxt < n_chunks))
            def _recycle(pb=pb, p=p, nxt=nxt):
                scatter(p, pb, sem_s[pb]).wait()
                gather(nxt, pb, sem_g[pb]).start()

            gather(g, bb, sem_g[bb]).wait()

            @pl.loop(0, C)
            def _row(r, bb=bb):
                for j in range(D // LANES):
                    sl = pl.ds(j * LANES, LANES)
                    rows_v[bb, r, sl] = rows_v[bb, r, sl] * SCALE

            scatter(g, bb, sem_s[bb]).start()

    for b in range(NBUF):
        scatter(n_chunks - NBUF + b, b, sem_s[b]).wait()


def kernel(x, table):
    batch, seq = x.shape
    b_total = batch * seq
    n_chunks = b_total // (NW * C)
    x_parts = x.reshape(NW, n_chunks, C).astype(jnp.int32)

    mesh = plsc.VectorSubcoreMesh(
        core_axis_name="c", subcore_axis_name="s", num_cores=NC,
        num_subcores=NS)
    out = pl.kernel(
        _emb_body,
        out_type=jax.ShapeDtypeStruct((b_total, D), jnp.float32),
        mesh=mesh,
        scratch_types=[
            pltpu.VMEM((n_chunks, C), jnp.int32),
            pltpu.VMEM((NBUF, C, D), jnp.float32),
            *([pltpu.SemaphoreType.DMA] * (2 * NBUF)),
        ],
        compiler_params=pltpu.CompilerParams(use_tc_tiling_on_sc=False),
    )(x_parts, table)
    return out.reshape(batch, seq, D)


# speedup vs baseline: 1.1449x; 1.1449x over previous
import math

import jax
import jax.numpy as jnp
from jax import lax
from jax.experimental import pallas as pl
from jax.experimental.pallas import tpu as pltpu
from jax.experimental.pallas import tpu_sc as plsc

D = 64
SCALE = math.sqrt(D)
NC = 2
NS = 16
NW = NC * NS
C = 128
NBUF = 4
LANES = 16


def _emb_body(x_hbm, table_hbm, out_hbm, idx_v, rows_v, *sems):
    sem_g = sems[:NBUF]
    sem_s = sems[NBUF:]
    n_chunks = x_hbm.shape[1]
    b_per_w = n_chunks * C
    wid = lax.axis_index("s") * NC + lax.axis_index("c")
    base = wid * b_per_w

    def out_slice(g):
        return out_hbm.at[pl.ds(base + g * C, C)]

    def gather(g, b, sem):
        return pltpu.make_async_copy(
            table_hbm.at[idx_v.at[g]], rows_v.at[b], sem)

    def scatter(g, b, sem):
        return pltpu.make_async_copy(rows_v.at[b], out_slice(g), sem)

    pltpu.sync_copy(x_hbm.at[wid], idx_v)

    for b in range(NBUF):
        gather(b, b, sem_g[b]).start()

    @pl.loop(0, n_chunks // NBUF)
    def _outer(t):
        g0 = t * NBUF
        for bb in range(NBUF):
            g = g0 + bb
            pb = (bb - 1) % NBUF
            p = g - 1
            nxt = p + NBUF

            @pl.when(jnp.logical_and(p >= 0, nxt < n_chunks))
            def _recycle(pb=pb, p=p, nxt=nxt):
                scatter(p, pb, sem_s[pb]).wait()
                gather(nxt, pb, sem_g[pb]).start()

            gather(g, bb, sem_g[bb]).wait()

            @pl.loop(0, C)
            def _row(r, bb=bb):
                for j in range(D // LANES):
                    sl = pl.ds(j * LANES, LANES)
                    rows_v[bb, r, sl] = rows_v[bb, r, sl] * SCALE

            scatter(g, bb, sem_s[bb]).start()

    for b in range(NBUF):
        scatter(n_chunks - NBUF + b, b, sem_s[b]).wait()


def kernel(x, table):
    batch, seq = x.shape
    b_total = batch * seq
    n_chunks = b_total // (NW * C)
    x_parts = x.reshape(NW, n_chunks, C).astype(jnp.int32)

    mesh = plsc.VectorSubcoreMesh(
        core_axis_name="c", subcore_axis_name="s", num_cores=NC,
        num_subcores=NS)
    out = pl.kernel(
        _emb_body,
        out_type=jax.ShapeDtypeStruct((b_total, D), jnp.float32),
        mesh=mesh,
        scratch_types=[
            pltpu.VMEM((n_chunks, C), jnp.int32),
            pltpu.VMEM((NBUF, C, D), jnp.float32),
            *([pltpu.SemaphoreType.DMA] * (2 * NBUF)),
        ],
        compiler_params=pltpu.CompilerParams(use_tc_tiling_on_sc=False),
    )(x_parts, table)
    return out.reshape(batch, seq, D)


# speedup vs baseline: 1.1479x; 1.0026x over previous
import math

import jax
import jax.numpy as jnp
from jax import lax
from jax.experimental import pallas as pl
from jax.experimental.pallas import tpu as pltpu
from jax.experimental.pallas import tpu_sc as plsc

D = 64
SCALE = math.sqrt(D)
NC = 2
NS = 16
NW = NC * NS
H = 100
NBUF = 4
LANES = 16


def _emb_body(x_hbm, table_hbm, out_hbm, idx_v, rows_v, *sems):
    sem_g0 = sems[:NBUF]
    sem_g1 = sems[NBUF:2 * NBUF]
    sem_s = sems[2 * NBUF:]
    seq = out_hbm.shape[1]
    rows_per_w = out_hbm.shape[0] // NW
    wid = lax.axis_index("s") * NC + lax.axis_index("c")
    base = wid * rows_per_w

    def gather(g, b, sem0, sem1):
        c0 = pltpu.make_async_copy(
            table_hbm.at[idx_v.at[2 * g]], rows_v.at[b, pl.ds(0, H)], sem0)
        c1 = pltpu.make_async_copy(
            table_hbm.at[idx_v.at[2 * g + 1]], rows_v.at[b, pl.ds(H, H)],
            sem1)
        return c0, c1

    def scatter(g, b, sem):
        return pltpu.make_async_copy(rows_v.at[b], out_hbm.at[base + g], sem)

    def gather_start(g, b):
        c0, c1 = gather(g, b, sem_g0[b], sem_g1[b])
        c0.start()
        c1.start()

    def gather_wait(g, b):
        c0, c1 = gather(g, b, sem_g0[b], sem_g1[b])
        c0.wait()
        c1.wait()

    pltpu.sync_copy(x_hbm.at[pl.ds(wid * 2 * rows_per_w, 2 * rows_per_w)],
                    idx_v)

    for b in range(NBUF):
        gather_start(b, b)

    @pl.loop(0, rows_per_w // NBUF)
    def _outer(t):
        g0 = t * NBUF
        for bb in range(NBUF):
            g = g0 + bb
            pb = (bb - 1) % NBUF
            p = g - 1
            nxt = p + NBUF

            @pl.when(jnp.logical_and(p >= 0, nxt < rows_per_w))
            def _recycle(pb=pb, p=p, nxt=nxt):
                scatter(p, pb, sem_s[pb]).wait()
                gather_start(nxt, pb)

            gather_wait(g, bb)

            @pl.loop(0, seq)
            def _row(r, bb=bb):
                for j in range(D // LANES):
                    sl = pl.ds(j * LANES, LANES)
                    rows_v[bb, r, sl] = rows_v[bb, r, sl] * SCALE

            scatter(g, bb, sem_s[bb]).start()

    for b in range(NBUF):
        scatter(rows_per_w - NBUF + b, b, sem_s[b]).wait()


def kernel(x, table):
    batch, seq = x.shape
    x2 = x.astype(jnp.int32).reshape(batch * seq // H, H)
    rows_per_w = batch // NW

    mesh = plsc.VectorSubcoreMesh(
        core_axis_name="c", subcore_axis_name="s", num_cores=NC,
        num_subcores=NS)
    out = pl.kernel(
        _emb_body,
        out_type=jax.ShapeDtypeStruct((batch, seq, D), jnp.float32),
        mesh=mesh,
        scratch_types=[
            pltpu.VMEM((2 * rows_per_w, H), jnp.int32),
            pltpu.VMEM((NBUF, seq, D), jnp.float32),
            *([pltpu.SemaphoreType.DMA] * (3 * NBUF)),
        ],
        compiler_params=pltpu.CompilerParams(use_tc_tiling_on_sc=False),
    )(x2, table)
    return out
